# Initial kernel scaffold; baseline (speedup 1.0000x reference)
import functools

import jax
import jax.numpy as jnp
from jax import lax
from jax.experimental import pallas as pl
from jax.experimental.pallas import tpu as pltpu
from jax.experimental.pallas import tpu_sc as plsc

EMBED_DIM = 64
NUM_WORKERS = 32
TOTAL = 4096 * 50
B_PER_W = TOTAL // NUM_WORKERS
CHUNK = 800
NCHUNK = B_PER_W // CHUNK

_mesh = plsc.VectorSubcoreMesh(core_axis_name="c", subcore_axis_name="s")


@functools.partial(
    pl.kernel,
    mesh=_mesh,
    out_type=jax.ShapeDtypeStruct((TOTAL, EMBED_DIM), jnp.float32),
    scratch_types=[
        pltpu.VMEM((CHUNK,), jnp.int32),
        pltpu.VMEM((CHUNK, EMBED_DIM), jnp.float32),
        pltpu.SemaphoreType.DMA,
    ],
)
def _gather_kernel(idx_hbm, table_hbm, out_hbm, idx_v, rows_v, sem):
    wid = lax.axis_index("s") * 2 + lax.axis_index("c")
    base = wid * B_PER_W
    for i in range(NCHUNK):
        cb = base + i * CHUNK
        pltpu.sync_copy(idx_hbm.at[pl.ds(cb, CHUNK)], idx_v)
        pltpu.async_copy(table_hbm.at[idx_v], rows_v, sem).wait()
        pltpu.sync_copy(rows_v, out_hbm.at[pl.ds(cb, CHUNK)])


def kernel(indices, table):
    idx = indices.reshape(-1).astype(jnp.int32)
    out = _gather_kernel(idx, table)
    return out.reshape(indices.shape + (table.shape[-1],))


# speedup vs baseline: 4.5456x; 4.5456x over previous
import functools

import jax
import jax.numpy as jnp
from jax import lax
from jax.experimental import pallas as pl
from jax.experimental.pallas import tpu as pltpu
from jax.experimental.pallas import tpu_sc as plsc

EMBED_DIM = 64
NUM_WORKERS = 32
TOTAL = 4096 * 50
B_PER_W = TOTAL // NUM_WORKERS
CHUNK = 800
NCHUNK = B_PER_W // CHUNK

_mesh = plsc.VectorSubcoreMesh(core_axis_name="c", subcore_axis_name="s")


@functools.partial(
    pl.kernel,
    mesh=_mesh,
    out_type=jax.ShapeDtypeStruct((TOTAL, EMBED_DIM), jnp.float32),
    scratch_types=[
        pltpu.VMEM((CHUNK,), jnp.int32),
        pltpu.VMEM((CHUNK, EMBED_DIM), jnp.float32),
        pltpu.SemaphoreType.DMA,
    ],
    compiler_params=pltpu.CompilerParams(use_tc_tiling_on_sc=False),
)
def _gather_kernel(idx_hbm, table_hbm, out_hbm, idx_v, rows_v, sem):
    wid = lax.axis_index("s") * 2 + lax.axis_index("c")
    base = wid * B_PER_W
    for i in range(NCHUNK):
        cb = base + i * CHUNK
        pltpu.sync_copy(idx_hbm.at[pl.ds(cb, CHUNK)], idx_v)
        pltpu.async_copy(table_hbm.at[idx_v], rows_v, sem).wait()
        pltpu.sync_copy(rows_v, out_hbm.at[pl.ds(cb, CHUNK)])


def kernel(indices, table):
    idx = indices.reshape(-1).astype(jnp.int32)
    out = _gather_kernel(idx, table)
    return out.reshape(indices.shape + (table.shape[-1],))


# speedup vs baseline: 4.6142x; 1.0151x over previous
import functools

import jax
import jax.numpy as jnp
from jax import lax
from jax.experimental import pallas as pl
from jax.experimental.pallas import tpu as pltpu
from jax.experimental.pallas import tpu_sc as plsc

EMBED_DIM = 64
NUM_WORKERS = 32
TOTAL = 4096 * 50
B_PER_W = TOTAL // NUM_WORKERS
CHUNK = 800
NCHUNK = B_PER_W // CHUNK

_mesh = plsc.VectorSubcoreMesh(core_axis_name="c", subcore_axis_name="s")


@functools.partial(
    pl.kernel,
    mesh=_mesh,
    out_type=jax.ShapeDtypeStruct((TOTAL, EMBED_DIM), jnp.float32),
    scratch_types=[
        pltpu.VMEM((B_PER_W,), jnp.int32),
        pltpu.VMEM((CHUNK, EMBED_DIM), jnp.float32),
        pltpu.VMEM((CHUNK, EMBED_DIM), jnp.float32),
        pltpu.SemaphoreType.DMA,
        pltpu.SemaphoreType.DMA,
    ],
    compiler_params=pltpu.CompilerParams(use_tc_tiling_on_sc=False),
)
def _gather_kernel(idx_hbm, table_hbm, out_hbm, idx_v, rows_a, rows_b, gsem, osem):
    wid = lax.axis_index("s") * 2 + lax.axis_index("c")
    base = wid * B_PER_W
    bufs = (rows_a, rows_b)

    pltpu.sync_copy(idx_hbm.at[pl.ds(base, B_PER_W)], idx_v)

    gather_cp = [None] * NCHUNK
    out_cp = [None] * NCHUNK

    def start_gather(i):
        gather_cp[i] = pltpu.async_copy(
            table_hbm.at[idx_v.at[pl.ds(i * CHUNK, CHUNK)]],
            bufs[i % 2],
            gsem,
        )

    start_gather(0)
    for i in range(NCHUNK):
        gather_cp[i].wait()
        if i + 1 < NCHUNK:
            if i >= 1:
                out_cp[i - 1].wait()
            start_gather(i + 1)
        out_cp[i] = pltpu.async_copy(
            bufs[i % 2],
            out_hbm.at[pl.ds(base + i * CHUNK, CHUNK)],
            osem,
        )
    if NCHUNK >= 2:
        out_cp[NCHUNK - 2].wait()
    out_cp[NCHUNK - 1].wait()


def kernel(indices, table):
    idx = indices.reshape(-1).astype(jnp.int32)
    out = _gather_kernel(idx, table)
    return out.reshape(indices.shape + (table.shape[-1],))


# speedup vs baseline: 4.6716x; 1.0124x over previous
import functools

import jax
import jax.numpy as jnp
from jax import lax
from jax.experimental import pallas as pl
from jax.experimental.pallas import tpu as pltpu
from jax.experimental.pallas import tpu_sc as plsc

EMBED_DIM = 64
NUM_WORKERS = 32
TOTAL = 4096 * 50
B_PER_W = TOTAL // NUM_WORKERS
CHUNK = 400
NCHUNK = B_PER_W // CHUNK
NBUF = 4

_mesh = plsc.VectorSubcoreMesh(core_axis_name="c", subcore_axis_name="s")


@functools.partial(
    pl.kernel,
    mesh=_mesh,
    out_type=jax.ShapeDtypeStruct((TOTAL, EMBED_DIM), jnp.float32),
    scratch_types=(
        [pltpu.VMEM((B_PER_W,), jnp.int32)]
        + [pltpu.VMEM((CHUNK, EMBED_DIM), jnp.float32) for _ in range(NBUF)]
        + [pltpu.SemaphoreType.DMA for _ in range(2 * NBUF)]
    ),
    compiler_params=pltpu.CompilerParams(use_tc_tiling_on_sc=False),
)
def _gather_kernel(idx_hbm, table_hbm, out_hbm, idx_v, *scratch):
    bufs = scratch[:NBUF]
    gsems = scratch[NBUF:2 * NBUF]
    osems = scratch[2 * NBUF:]
    wid = lax.axis_index("s") * 2 + lax.axis_index("c")
    base = wid * B_PER_W

    pltpu.sync_copy(idx_hbm.at[pl.ds(base, B_PER_W)], idx_v)

    gather_cp = [None] * NCHUNK
    out_cp = [None] * NCHUNK

    def start_gather(i):
        gather_cp[i] = pltpu.async_copy(
            table_hbm.at[idx_v.at[pl.ds(i * CHUNK, CHUNK)]],
            bufs[i % NBUF],
            gsems[i % NBUF],
        )

    for i in range(NBUF - 1):
        start_gather(i)
    for i in range(NCHUNK):
        gather_cp[i].wait()
        out_cp[i] = pltpu.async_copy(
            bufs[i % NBUF],
            out_hbm.at[pl.ds(base + i * CHUNK, CHUNK)],
            osems[i % NBUF],
        )
        j = i + NBUF - 1
        if j < NCHUNK:
            if j - NBUF >= 0:
                out_cp[j - NBUF].wait()
            start_gather(j)
    for i in range(NCHUNK - NBUF, NCHUNK):
        if i >= 0 and out_cp[i] is not None:
            out_cp[i].wait()


def kernel(indices, table):
    idx = indices.reshape(-1).astype(jnp.int32)
    out = _gather_kernel(idx, table)
    return out.reshape(indices.shape + (table.shape[-1],))
